# baseline (speedup 1.0000x reference)
import jax
import jax.numpy as jnp
from jax import lax
from jax.experimental import pallas as pl
from jax.experimental.pallas import tpu as pltpu
from jax.experimental.pallas import tpu_sc as plsc

SEQ = 200
BATCH = 1024
D = 128
ROWS = SEQ * BATCH
NW = 32
ROWS_PER_W = ROWS // NW
CHUNK = 128
NCHUNK = ROWS_PER_W // CHUNK
NBUF = 2
NROUND = NCHUNK // NBUF
PE_ROWS = 16
LANES = 16
VPD = D // LANES
UNROLL = 8


def _embed_body(x_hbm, table_hbm, pe_hbm, out_hbm, idx_v, pe_v, *ring):
    bufs = ring[:NBUF]
    sems = ring[NBUF:2 * NBUF]
    wid = lax.axis_index("s") * 2 + lax.axis_index("c")
    base = wid * ROWS_PER_W
    s0 = lax.min((base // BATCH) // 8 * 8, SEQ - PE_ROWS)
    pltpu.sync_copy(x_hbm.at[wid], idx_v)

    def gather(c, b):
        pltpu.async_copy(table_hbm.at[idx_v.at[c]], bufs[b], sems[b])

    for b in range(NBUF):
        gather(b, b)
    pltpu.sync_copy(pe_hbm.at[pl.ds(s0, PE_ROWS)], pe_v)

    def add_pe(buf, s):
        pe_regs = [pe_v[s, pl.ds(LANES * j, LANES)] for j in range(VPD)]

        def row_body(r, carry):
            for u in range(UNROLL):
                for j in range(VPD):
                    plsc.addupdate(
                        buf.at[r * UNROLL + u, pl.ds(LANES * j, LANES)],
                        pe_regs[j])
            return carry

        lax.fori_loop(0, CHUNK // UNROLL, row_body, 0)

    def round_body(g, carry):
        for b in range(NBUF):
            c = g * NBUF + b
            start = base + c * CHUNK
            pltpu.make_async_copy(
                table_hbm.at[idx_v.at[c]], bufs[b], sems[b]).wait()
            add_pe(bufs[b], start // BATCH - s0)
            pltpu.async_copy(bufs[b], out_hbm.at[pl.ds(start, CHUNK)], sems[b])

        @pl.when(g < NROUND - 1)
        def _refill():
            for b in range(NBUF):
                c = (g + 1) * NBUF + b
                pltpu.make_async_copy(
                    bufs[b], out_hbm.at[pl.ds(base, CHUNK)], sems[b]).wait()
                gather(c, b)

        return carry

    lax.fori_loop(0, NROUND, round_body, 0)
    for b in range(NBUF):
        pltpu.make_async_copy(
            bufs[b], out_hbm.at[pl.ds(base, CHUNK)], sems[b]).wait()


def kernel(x, table, pe):
    xf = x.reshape(NW, NCHUNK, CHUNK).astype(jnp.int32)
    pef = pe.reshape(SEQ, D)
    mesh = plsc.VectorSubcoreMesh(core_axis_name="c", subcore_axis_name="s")
    out = pl.kernel(
        _embed_body,
        mesh=mesh,
        out_type=jax.ShapeDtypeStruct((ROWS, D), jnp.float32),
        scratch_types=(
            [pltpu.VMEM((NCHUNK, CHUNK), jnp.int32),
             pltpu.VMEM((PE_ROWS, D), jnp.float32)]
            + [pltpu.VMEM((CHUNK, D), jnp.float32) for _ in range(NBUF)]
            + [pltpu.SemaphoreType.DMA for _ in range(NBUF)]
        ),
    )(xf, table, pef)
    return out.reshape(SEQ, BATCH, D)


# speedup vs baseline: 1.0727x; 1.0727x over previous
import jax
import jax.numpy as jnp
from jax import lax
from jax.experimental import pallas as pl
from jax.experimental.pallas import tpu as pltpu
from jax.experimental.pallas import tpu_sc as plsc

SEQ = 200
BATCH = 1024
D = 128
ROWS = SEQ * BATCH
NW = 32
ROWS_PER_W = ROWS // NW
CHUNK = 128
NCHUNK = ROWS_PER_W // CHUNK
NBUF = 5
NROUND = NCHUNK // NBUF
PE_ROWS = 16
LANES = 16
VPD = D // LANES
UNROLL = 8


def _embed_body(x_hbm, table_hbm, pe_hbm, out_hbm, idx_v, pe_v, *ring):
    bufs = ring[:NBUF]
    sems = ring[NBUF:2 * NBUF]
    wid = lax.axis_index("s") * 2 + lax.axis_index("c")
    base = wid * ROWS_PER_W
    s0 = lax.min((base // BATCH) // 8 * 8, SEQ - PE_ROWS)
    pltpu.sync_copy(x_hbm.at[wid], idx_v)

    def gather(c, b):
        pltpu.async_copy(table_hbm.at[idx_v.at[c]], bufs[b], sems[b])

    for b in range(NBUF):
        gather(b, b)
    pltpu.sync_copy(pe_hbm.at[pl.ds(s0, PE_ROWS)], pe_v)

    def add_pe(buf, s):
        pe_regs = [pe_v[s, pl.ds(LANES * j, LANES)] for j in range(VPD)]

        def row_body(r, carry):
            for u in range(UNROLL):
                for j in range(VPD):
                    plsc.addupdate(
                        buf.at[r * UNROLL + u, pl.ds(LANES * j, LANES)],
                        pe_regs[j])
            return carry

        lax.fori_loop(0, CHUNK // UNROLL, row_body, 0)

    def round_body(g, carry):
        for b in range(NBUF):
            c = g * NBUF + b
            start = base + c * CHUNK
            pltpu.make_async_copy(
                table_hbm.at[idx_v.at[c]], bufs[b], sems[b]).wait()
            add_pe(bufs[b], start // BATCH - s0)
            pltpu.async_copy(
                bufs[b], out_hbm.at[start // BATCH, pl.ds(start % BATCH, CHUNK)],
                sems[b])

        @pl.when(g < NROUND - 1)
        def _refill():
            for b in range(NBUF):
                c = (g + 1) * NBUF + b
                pltpu.make_async_copy(
                    bufs[b], out_hbm.at[base // BATCH, pl.ds(0, CHUNK)],
                    sems[b]).wait()
                gather(c, b)

        return carry

    lax.fori_loop(0, NROUND, round_body, 0)
    for b in range(NBUF):
        pltpu.make_async_copy(
            bufs[b], out_hbm.at[base // BATCH, pl.ds(0, CHUNK)], sems[b]).wait()


def kernel(x, table, pe):
    xf = x.reshape(NW, NCHUNK, CHUNK).astype(jnp.int32)
    pef = pe.reshape(SEQ, D)
    mesh = plsc.VectorSubcoreMesh(core_axis_name="c", subcore_axis_name="s")
    out = pl.kernel(
        _embed_body,
        mesh=mesh,
        out_type=jax.ShapeDtypeStruct((SEQ, BATCH, D), jnp.float32),
        scratch_types=(
            [pltpu.VMEM((NCHUNK, CHUNK), jnp.int32),
             pltpu.VMEM((PE_ROWS, D), jnp.float32)]
            + [pltpu.VMEM((CHUNK, D), jnp.float32) for _ in range(NBUF)]
            + [pltpu.SemaphoreType.DMA for _ in range(NBUF)]
        ),
    )(xf, table, pef)
    return out


# speedup vs baseline: 1.0863x; 1.0127x over previous
import jax
import jax.numpy as jnp
from jax import lax
from jax.experimental import pallas as pl
from jax.experimental.pallas import tpu as pltpu
from jax.experimental.pallas import tpu_sc as plsc

SEQ = 200
BATCH = 1024
D = 128
ROWS = SEQ * BATCH
NW = 32
ROWS_PER_W = ROWS // NW
CHUNK = 128
NCHUNK = ROWS_PER_W // CHUNK
NBUF = 5
NROUND = NCHUNK // NBUF
PE_ROWS = 16
LANES = 16
VPD = D // LANES
UNROLL = 8


def _embed_body(x_hbm, table_hbm, pe_hbm, out_hbm, idx_v, pe_v, *ring):
    bufs = ring[:NBUF]
    sems = ring[NBUF:2 * NBUF]
    wid = lax.axis_index("s") * 2 + lax.axis_index("c")
    base = wid * ROWS_PER_W
    s0 = lax.min((base // BATCH) // 8 * 8, SEQ - PE_ROWS)
    pltpu.sync_copy(x_hbm.at[wid], idx_v)

    def gather(c, b):
        pltpu.async_copy(table_hbm.at[idx_v.at[c]], bufs[b], sems[b])

    for b in range(NBUF):
        gather(b, b)
    pltpu.sync_copy(pe_hbm.at[pl.ds(s0, PE_ROWS)], pe_v)

    def add_pe(buf, s):
        pe_regs = [pe_v[s, pl.ds(LANES * j, LANES)] for j in range(VPD)]

        def row_body(r, carry):
            for u in range(UNROLL):
                for j in range(VPD):
                    plsc.addupdate(
                        buf.at[r * UNROLL + u, pl.ds(LANES * j, LANES)],
                        pe_regs[j])
            return carry

        lax.fori_loop(0, CHUNK // UNROLL, row_body, 0)

    def round_body(g, carry):
        for b in range(NBUF):
            c = g * NBUF + b
            start = base + c * CHUNK
            pltpu.make_async_copy(
                table_hbm.at[idx_v.at[c]], bufs[b], sems[b]).wait()
            add_pe(bufs[b], start // BATCH - s0)
            pltpu.async_copy(
                bufs[b], out_hbm.at[start // BATCH, pl.ds(start % BATCH, CHUNK)],
                sems[b])

            if b > 0:
                @pl.when(g < NROUND - 1)
                def _refill_prev(b=b):
                    pltpu.make_async_copy(
                        bufs[b - 1], out_hbm.at[base // BATCH, pl.ds(0, CHUNK)],
                        sems[b - 1]).wait()
                    gather((g + 1) * NBUF + b - 1, b - 1)

        @pl.when(g < NROUND - 1)
        def _refill_last():
            pltpu.make_async_copy(
                bufs[NBUF - 1], out_hbm.at[base // BATCH, pl.ds(0, CHUNK)],
                sems[NBUF - 1]).wait()
            gather((g + 1) * NBUF + NBUF - 1, NBUF - 1)

        return carry

    lax.fori_loop(0, NROUND, round_body, 0)
    for b in range(NBUF):
        pltpu.make_async_copy(
            bufs[b], out_hbm.at[base // BATCH, pl.ds(0, CHUNK)], sems[b]).wait()


def kernel(x, table, pe):
    xf = x.reshape(NW, NCHUNK, CHUNK).astype(jnp.int32)
    pef = pe.reshape(SEQ, D)
    mesh = plsc.VectorSubcoreMesh(core_axis_name="c", subcore_axis_name="s")
    out = pl.kernel(
        _embed_body,
        mesh=mesh,
        out_type=jax.ShapeDtypeStruct((SEQ, BATCH, D), jnp.float32),
        scratch_types=(
            [pltpu.VMEM((NCHUNK, CHUNK), jnp.int32),
             pltpu.VMEM((PE_ROWS, D), jnp.float32)]
            + [pltpu.VMEM((CHUNK, D), jnp.float32) for _ in range(NBUF)]
            + [pltpu.SemaphoreType.DMA for _ in range(NBUF)]
        ),
    )(xf, table, pef)
    return out
